# baseline (speedup 1.0000x reference)
import functools

import jax
import jax.numpy as jnp
from jax.experimental import pallas as pl

_KS = 4
_NBIN = _KS * _KS * _KS
_EXTENT = float(1.5 * 6 * 0.025)


def _mm(x, w):
    n, k = x.shape
    o = w.shape[1]
    bn = 512
    n_pad = (n + bn - 1) // bn * bn
    if n_pad != n:
        x = jnp.pad(x, ((0, n_pad - n), (0, 0)))

    def body(x_ref, w_ref, o_ref):
        o_ref[...] = jnp.dot(x_ref[...], w_ref[...],
                             preferred_element_type=jnp.float32)

    out = pl.pallas_call(
        body,
        grid=(n_pad // bn,),
        in_specs=[pl.BlockSpec((bn, k), lambda i: (i, 0)),
                  pl.BlockSpec((k, o), lambda i: (0, 0))],
        out_specs=pl.BlockSpec((bn, o), lambda i: (i, 0)),
        out_shape=jax.ShapeDtypeStruct((n_pad, o), jnp.float32),
    )(x, w)
    return out[:n]


def _edge_geom(pos_src, pos_dst, edges):
    src, dst = edges[0], edges[1]
    rel = pos_src[src] - pos_dst[dst]
    radius = _EXTENT / 2.0
    p = rel / radius
    r2 = jnp.sum(p * p, axis=-1)
    win = jnp.clip((1.0 - r2) ** 3, 0.0, 1.0)
    norm2 = jnp.sqrt(jnp.maximum(r2, 1e-16))
    norminf = jnp.max(jnp.abs(p), axis=-1)
    scale = jnp.where(norminf > 1e-8, norm2 / jnp.maximum(norminf, 1e-8), 0.0)
    q = jnp.clip(p * scale[:, None], -1.0, 1.0)
    g = (q + 1.0) * 0.5 * (_KS - 1)
    g0f = jnp.clip(jnp.floor(g), 0.0, _KS - 2)
    frac = g - g0f
    g0 = g0f.astype(jnp.int32)
    return win, frac, g0


def _scatter_bins(feat_src, src, dst, win, frac, g0, n_dst):
    in_ch = feat_src.shape[-1]
    fsrc = feat_src[src]
    S = jnp.zeros((n_dst * _NBIN, in_ch), dtype=feat_src.dtype)
    for cx in (0, 1):
        for cy in (0, 1):
            for cz in (0, 1):
                wx = frac[:, 0] if cx else (1.0 - frac[:, 0])
                wy = frac[:, 1] if cy else (1.0 - frac[:, 1])
                wz = frac[:, 2] if cz else (1.0 - frac[:, 2])
                w = win * wx * wy * wz
                bin_idx = ((g0[:, 0] + cx) * (_KS * _KS)
                           + (g0[:, 1] + cy) * _KS + (g0[:, 2] + cz))
                flat = dst * _NBIN + bin_idx
                S = S.at[flat].add(w[:, None] * fsrc)
    return S.reshape(n_dst, _NBIN * in_ch)


def kernel(pos, vel, box, box_feats, edge_index_fluid, edge_index_obstacle,
           W0f, b0f, W0o, b0o, D0w, D0b, W1, b1, D1w, D1b,
           W2, b2, D2w, D2b, W3, b3, D3w, D3b):
    n = pos.shape[0]
    fluid_feats = jnp.concatenate([jnp.ones_like(pos[:, 0:1]), vel], axis=-1)

    winf, fracf, g0f_ = _edge_geom(pos, pos, edge_index_fluid)
    wino, fraco, g0o_ = _edge_geom(box, pos, edge_index_obstacle)
    sf, df = edge_index_fluid[0], edge_index_fluid[1]
    so, do = edge_index_obstacle[0], edge_index_obstacle[1]

    S0f = _scatter_bins(fluid_feats, sf, df, winf, fracf, g0f_, n)
    a0f = _mm(S0f, W0f.reshape(_NBIN * 4, 32)) + b0f
    S0o = _scatter_bins(box_feats, so, do, wino, fraco, g0o_, n)
    a0o = _mm(S0o, W0o.reshape(_NBIN * 3, 32)) + b0o
    a0d = _mm(fluid_feats, D0w) + D0b
    feats = jnp.concatenate([a0o, a0f, a0d], axis=-1)

    h1 = jax.nn.relu(feats)
    S1 = _scatter_bins(h1, sf, df, winf, fracf, g0f_, n)
    out1 = _mm(S1, W1.reshape(_NBIN * 96, 64)) + b1 + _mm(h1, D1w) + D1b

    h2 = jax.nn.relu(out1)
    S2 = _scatter_bins(h2, sf, df, winf, fracf, g0f_, n)
    out2 = (_mm(S2, W2.reshape(_NBIN * 64, 64)) + b2
            + _mm(h2, D2w) + D2b + out1)
    return out2


# speedup vs baseline: 3.8691x; 3.8691x over previous
import functools

import jax
import jax.numpy as jnp
from jax import lax
from jax.experimental import pallas as pl
from jax.experimental.pallas import tpu as pltpu
from jax.experimental.pallas import tpu_sc as plsc

_KS = 4
_NBIN = 64
_EXTENT = float(1.5 * 6 * 0.025)
_INV_RADIUS = 2.0 / _EXTENT
_NPAD = 10240
_NTILE = 16
_CHUNK_DST = 256
_NCHUNK = 20
_HALF = _CHUNK_DST * _NCHUNK
_EBLK = 2000

_CP = pltpu.CompilerParams(needs_layout_passes=False,
                           use_tc_tiling_on_sc=False)

_CORNERS = [(cx, cy, cz) for cx in (0, 1) for cy in (0, 1) for cz in (0, 1)]


def _rsqrt(x):
    b = plsc.bitcast(x, jnp.int32)
    y = plsc.bitcast(jnp.int32(0x5F3759DF) - (b >> 1), jnp.float32)
    y = y * (1.5 - 0.5 * x * y * y)
    y = y * (1.5 - 0.5 * x * y * y)
    y = y * (1.5 - 0.5 * x * y * y)
    return y


def _make_sc_scatter(n_edges, in_ch):
    assert in_ch % 16 == 0
    kblk = in_ch // 16
    acols = in_ch + 16
    ept = n_edges // _NTILE
    nblk = ept // _EBLK
    nvec = _EBLK // 16
    assert ept * _NTILE == n_edges and nblk * _EBLK == ept and nvec * 16 == _EBLK
    srows = _CHUNK_DST * _NBIN
    trows = srows // _NTILE
    mesh = plsc.VectorSubcoreMesh(core_axis_name="c", subcore_axis_name="s")

    @functools.partial(
        pl.kernel, mesh=mesh, compiler_params=_CP,
        out_type=jax.ShapeDtypeStruct((_NPAD * _NBIN, in_ch), jnp.float32),
        scratch_types=[
            pltpu.VMEM_SHARED((srows, in_ch), jnp.float32),
            pltpu.VMEM((_EBLK,), jnp.int32),
            pltpu.VMEM((_EBLK,), jnp.int32),
            pltpu.VMEM((_CHUNK_DST,), jnp.float32),
            pltpu.VMEM((_CHUNK_DST,), jnp.float32),
            pltpu.VMEM((_CHUNK_DST,), jnp.float32),
            pltpu.VMEM((128, in_ch), jnp.float32),
            pltpu.VMEM((128,), jnp.int32),
            pltpu.VMEM((16, acols), jnp.float32),
            pltpu.VMEM((48,), jnp.int32),
            pltpu.VMEM((48,), jnp.int32),
            pltpu.SemaphoreType.DMA,
        ],
    )
    def sc_kernel(esrc, edst, qx, qy, qz, feats, s_hbm,
                  shS, ebs, ebd, qcx, qcy, qcz, stage, idxm, fbuf,
                  pends, pendd, sem):
        cid = lax.axis_index("c")
        sid = lax.axis_index("s")
        ii = lax.iota(jnp.int32, 16)

        def zero_stage_and_slice():
            z = ii.astype(jnp.float32) * 0.0
            for r in range(128):
                for k in range(kblk):
                    stage[r, pl.ds(k * 16, 16)] = z
            for j in range(trows // 128):
                pltpu.sync_copy(stage, shS.at[pl.ds(sid * trows + j * 128, 128)])

        def process(cntv, lo):
            mask = ii < cntv
            sv = jnp.where(mask, pends[pl.ds(0, 16)], 0)
            dv = jnp.where(mask, pendd[pl.ds(0, 16)], lo)
            pltpu.async_copy(feats.at[sv], fbuf, sem).wait()
            sx = plsc.load_gather(fbuf, [ii, ii * 0 + in_ch])
            sy = plsc.load_gather(fbuf, [ii, ii * 0 + (in_ch + 1)])
            sz = plsc.load_gather(fbuf, [ii, ii * 0 + (in_ch + 2)])
            dloc = dv - lo
            dx = plsc.load_gather(qcx, [dloc])
            dy = plsc.load_gather(qcy, [dloc])
            dz = plsc.load_gather(qcz, [dloc])
            pxv = (sx - dx) * _INV_RADIUS
            pyv = (sy - dy) * _INV_RADIUS
            pzv = (sz - dz) * _INV_RADIUS
            r2 = pxv * pxv + pyv * pyv + pzv * pzv
            win = 1.0 - r2
            win = win * win * win
            win = jnp.clip(win, 0.0, 1.0)
            win = jnp.where(mask, win, 0.0)
            r2c = jnp.maximum(r2, 1e-16)
            norm2 = r2c * _rsqrt(r2c)
            ninf = jnp.maximum(jnp.maximum(jnp.abs(pxv), jnp.abs(pyv)),
                               jnp.abs(pzv))
            scale = jnp.where(ninf > 1e-8,
                              norm2 / jnp.maximum(ninf, 1e-8), 0.0)
            g0 = []
            fr = []
            for pv in (pxv, pyv, pzv):
                g = (jnp.clip(pv * scale, -1.0, 1.0) + 1.0) * (0.5 * (_KS - 1))
                gi = jnp.clip(g.astype(jnp.int32), 0, _KS - 2)
                g0.append(gi)
                fr.append(g - gi.astype(jnp.float32))
            rowbase = dloc * _NBIN + g0[0] * 16 + g0[1] * 4 + g0[2]
            wx = (win * (1.0 - fr[0]), win * fr[0])
            wy = (1.0 - fr[1], fr[1])
            wz = (1.0 - fr[2], fr[2])
            wcs = []
            for c, (cx, cy, cz) in enumerate(_CORNERS):
                wcs.append(wx[cx] * wy[cy] * wz[cz])
                idxm[pl.ds(c * 16, 16)] = rowbase + (cx * 16 + cy * 4 + cz)
            for e in range(16):
                fe = [fbuf[e, pl.ds(k * 16, 16)] for k in range(kblk)]
                for c in range(8):
                    w_sc = wcs[c][e]
                    for k in range(kblk):
                        stage[c * 16 + e, pl.ds(k * 16, 16)] = fe[k] * w_sc
            pltpu.sync_copy(stage, shS.at[idxm], add=True)
            for a in (pends, pendd):
                a[pl.ds(0, 16)] = a[pl.ds(16, 16)]
                a[pl.ds(16, 16)] = a[pl.ds(32, 16)]
            return cntv - 16

        zero_stage_and_slice()
        plsc.subcore_barrier()

        def chunk_body(ch, carry):
            lo = cid * _HALF + ch * _CHUNK_DST
            pltpu.sync_copy(qx.at[pl.ds(lo, _CHUNK_DST)], qcx)
            pltpu.sync_copy(qy.at[pl.ds(lo, _CHUNK_DST)], qcy)
            pltpu.sync_copy(qz.at[pl.ds(lo, _CHUNK_DST)], qcz)

            def block_body(b, cnt):
                base = sid * ept + b * _EBLK
                pltpu.sync_copy(esrc.at[pl.ds(base, _EBLK)], ebs)
                pltpu.sync_copy(edst.at[pl.ds(base, _EBLK)], ebd)

                def scan_body(i, cnt):
                    sv = ebs[pl.ds(i * 16, 16)]
                    dv = ebd[pl.ds(i * 16, 16)]
                    m = (dv >= lo) & (dv < lo + _CHUNK_DST)
                    plsc.store_compressed(pends.at[pl.ds(cnt, 16)], sv, mask=m)
                    plsc.store_compressed(pendd.at[pl.ds(cnt, 16)], dv, mask=m)
                    cnt = cnt + plsc.all_reduce_population_count(m)[0]
                    return lax.cond(cnt >= 16,
                                    lambda c: process(16, lo) + (c - 16),
                                    lambda c: c, cnt)

                return lax.fori_loop(0, nvec, scan_body, cnt)

            cnt = lax.fori_loop(0, nblk, block_body, 0)
            cnt = lax.cond(cnt > 0, lambda c: process(c, lo) * 0,
                           lambda c: c, cnt)
            plsc.subcore_barrier()
            pltpu.sync_copy(
                shS.at[pl.ds(sid * trows, trows)],
                s_hbm.at[pl.ds(lo * _NBIN + sid * trows, trows)])
            zero_stage_and_slice()
            plsc.subcore_barrier()
            return carry

        lax.fori_loop(0, _NCHUNK, chunk_body, 0)

    return sc_kernel


_sc_fluid16 = _make_sc_scatter(320000, 16)
_sc_obst16 = _make_sc_scatter(160000, 16)
_sc_fluid96 = _make_sc_scatter(320000, 96)
_sc_fluid64 = _make_sc_scatter(320000, 64)


def _tc_layer0(S0o, W0o, S0f, W0f, ff, D0w, bcat):
    bn = 256
    ko = W0o.shape[0]
    kf = W0f.shape[0]

    def body(so_ref, wo_ref, sf_ref, wf_ref, ff_ref, dw_ref, b_ref, o_ref):
        a0o = jnp.dot(so_ref[...], wo_ref[...],
                      preferred_element_type=jnp.float32)
        a0f = jnp.dot(sf_ref[...], wf_ref[...],
                      preferred_element_type=jnp.float32)
        a0d = jnp.dot(ff_ref[...], dw_ref[...],
                      preferred_element_type=jnp.float32)
        x = jnp.concatenate([a0o, a0f, a0d], axis=1) + b_ref[...]
        o_ref[...] = jnp.maximum(x, 0.0)

    return pl.pallas_call(
        body,
        grid=(_NPAD // bn,),
        in_specs=[pl.BlockSpec((bn, ko), lambda i: (i, 0)),
                  pl.BlockSpec((ko, 32), lambda i: (0, 0)),
                  pl.BlockSpec((bn, kf), lambda i: (i, 0)),
                  pl.BlockSpec((kf, 32), lambda i: (0, 0)),
                  pl.BlockSpec((bn, 16), lambda i: (i, 0)),
                  pl.BlockSpec((16, 32), lambda i: (0, 0)),
                  pl.BlockSpec((1, 96), lambda i: (0, 0))],
        out_specs=pl.BlockSpec((bn, 96), lambda i: (i, 0)),
        out_shape=jax.ShapeDtypeStruct((_NPAD, 96), jnp.float32),
    )(S0o, W0o, S0f, W0f, ff, D0w, bcat)


def _tc_layer(S2d, Wflat, h, Dw, bias, res, want_relu):
    bn = 256
    k = Wflat.shape[0]
    kh = Dw.shape[0]
    oc = Wflat.shape[1]
    n_out = 2 if want_relu else 1

    def body(*refs):
        if res is not None:
            s_ref, w_ref, h_ref, dw_ref, b_ref, r_ref = refs[:6]
            outs = refs[6:]
        else:
            s_ref, w_ref, h_ref, dw_ref, b_ref = refs[:5]
            outs = refs[5:]
        x = jnp.dot(s_ref[...], w_ref[...],
                    preferred_element_type=jnp.float32)
        x = x + jnp.dot(h_ref[...], dw_ref[...],
                        preferred_element_type=jnp.float32)
        x = x + b_ref[...]
        if res is not None:
            x = x + r_ref[...]
        outs[0][...] = x
        if want_relu:
            outs[1][...] = jnp.maximum(x, 0.0)

    in_specs = [pl.BlockSpec((bn, k), lambda i: (i, 0)),
                pl.BlockSpec((k, oc), lambda i: (0, 0)),
                pl.BlockSpec((bn, kh), lambda i: (i, 0)),
                pl.BlockSpec((kh, oc), lambda i: (0, 0)),
                pl.BlockSpec((1, oc), lambda i: (0, 0))]
    args = [S2d, Wflat, h, Dw, bias]
    if res is not None:
        in_specs.append(pl.BlockSpec((bn, oc), lambda i: (i, 0)))
        args.append(res)
    out_specs = [pl.BlockSpec((bn, oc), lambda i: (i, 0))] * n_out
    out_shape = [jax.ShapeDtypeStruct((_NPAD, oc), jnp.float32)] * n_out
    if n_out == 1:
        out_specs, out_shape = out_specs[0], out_shape[0]
    return pl.pallas_call(
        body,
        grid=(_NPAD // bn,),
        in_specs=in_specs,
        out_specs=out_specs,
        out_shape=out_shape,
    )(*args)


def kernel(pos, vel, box, box_feats, edge_index_fluid, edge_index_obstacle,
           W0f, b0f, W0o, b0o, D0w, D0b, W1, b1, D1w, D1b,
           W2, b2, D2w, D2b, W3, b3, D3w, D3b):
    n = pos.shape[0]
    m = box.shape[0]
    f32 = jnp.float32

    posp = jnp.pad(pos, ((0, _NPAD - n), (0, 0)))
    qx, qy, qz = posp[:, 0], posp[:, 1], posp[:, 2]
    post16 = jnp.pad(posp, ((0, 0), (0, 13)))
    boxt16 = jnp.pad(box, ((0, _NPAD - m), (0, 13)))
    sf, df = edge_index_fluid[0], edge_index_fluid[1]
    so, do = edge_index_obstacle[0], edge_index_obstacle[1]

    ff = jnp.concatenate([jnp.ones_like(vel[:, 0:1]), vel], axis=-1)
    ff16 = jnp.pad(ff, ((0, _NPAD - n), (0, 12)))
    bf16 = jnp.pad(box_feats, ((0, _NPAD - m), (0, 13)))
    ff_aug = jnp.concatenate([ff16, post16], axis=1)
    bf_aug = jnp.concatenate([bf16, boxt16], axis=1)

    W0f_p = jnp.pad(W0f.reshape(_NBIN, 4, 32),
                    ((0, 0), (0, 12), (0, 0))).reshape(_NBIN * 16, 32)
    W0o_p = jnp.pad(W0o.reshape(_NBIN, 3, 32),
                    ((0, 0), (0, 13), (0, 0))).reshape(_NBIN * 16, 32)
    W1f = W1.reshape(_NBIN * 96, 64)
    W2f = W2.reshape(_NBIN * 64, 64)

    S0f = _sc_fluid16(sf, df, qx, qy, qz, ff_aug)
    S0o = _sc_obst16(so, do, qx, qy, qz, bf_aug)
    bcat = jnp.concatenate([b0o, b0f, D0b]).reshape(1, 96)
    h1 = _tc_layer0(S0o.reshape(_NPAD, _NBIN * 16), W0o_p,
                    S0f.reshape(_NPAD, _NBIN * 16), W0f_p,
                    ff16, jnp.pad(D0w, ((0, 12), (0, 0))), bcat)

    h1_aug = jnp.concatenate([h1, post16], axis=1)
    S1 = _sc_fluid96(sf, df, qx, qy, qz, h1_aug)
    out1, h2 = _tc_layer(S1.reshape(_NPAD, _NBIN * 96), W1f, h1, D1w,
                         (b1 + D1b).reshape(1, 64), None, True)

    h2_aug = jnp.concatenate([h2, post16], axis=1)
    S2 = _sc_fluid64(sf, df, qx, qy, qz, h2_aug)
    out2 = _tc_layer(S2.reshape(_NPAD, _NBIN * 64), W2f, h2, D2w,
                     (b2 + D2b).reshape(1, 64), out1, False)
    return out2[:n]


# speedup vs baseline: 3.9940x; 1.0323x over previous
import functools

import jax
import jax.numpy as jnp
from jax import lax
from jax.experimental import pallas as pl
from jax.experimental.pallas import tpu as pltpu
from jax.experimental.pallas import tpu_sc as plsc

_KS = 4
_NBIN = 64
_EXTENT = float(1.5 * 6 * 0.025)
_INV_RADIUS = 2.0 / _EXTENT
_NPAD = 10240
_NTILE = 16
_CHUNK_DST = 256
_NCHUNK = 20
_HALF = _CHUNK_DST * _NCHUNK
_EBLK = 2000

_CP = pltpu.CompilerParams(needs_layout_passes=False,
                           use_tc_tiling_on_sc=False)

_CORNERS = [(cx, cy, cz) for cx in (0, 1) for cy in (0, 1) for cz in (0, 1)]


def _rsqrt(x):
    b = plsc.bitcast(x, jnp.int32)
    y = plsc.bitcast(jnp.int32(0x5F3759DF) - (b >> 1), jnp.float32)
    y = y * (1.5 - 0.5 * x * y * y)
    y = y * (1.5 - 0.5 * x * y * y)
    y = y * (1.5 - 0.5 * x * y * y)
    return y


def _make_sc_scatter(n_edges, in_ch, ebatch):
    assert in_ch % 16 == 0 and ebatch % 16 == 0
    kblk = in_ch // 16
    acols = in_ch + 16
    vb = ebatch // 16
    pcap = ebatch + 32
    ept = n_edges // _NTILE
    nblk = ept // _EBLK
    nvec = _EBLK // 16
    assert ept * _NTILE == n_edges and nblk * _EBLK == ept and nvec * 16 == _EBLK
    srows = _CHUNK_DST * _NBIN
    trows = srows // _NTILE
    mesh = plsc.VectorSubcoreMesh(core_axis_name="c", subcore_axis_name="s")

    @functools.partial(
        pl.kernel, mesh=mesh, compiler_params=_CP,
        out_type=jax.ShapeDtypeStruct((_NPAD * _NBIN, in_ch), jnp.float32),
        scratch_types=[
            pltpu.VMEM_SHARED((srows, in_ch), jnp.float32),
            pltpu.VMEM((_EBLK,), jnp.int32),
            pltpu.VMEM((_EBLK,), jnp.int32),
            pltpu.VMEM((_CHUNK_DST,), jnp.float32),
            pltpu.VMEM((_CHUNK_DST,), jnp.float32),
            pltpu.VMEM((_CHUNK_DST,), jnp.float32),
            pltpu.VMEM((vb * 128, in_ch), jnp.float32),
            [pltpu.VMEM((128,), jnp.int32) for _ in range(vb)],
            pltpu.VMEM((ebatch,), jnp.int32),
            pltpu.VMEM((ebatch, acols), jnp.float32),
            pltpu.VMEM((pcap,), jnp.int32),
            pltpu.VMEM((pcap,), jnp.int32),
            pltpu.SemaphoreType.DMA,
        ],
    )
    def sc_kernel(esrc, edst, qx, qy, qz, feats, s_hbm,
                  shS, ebs, ebd, qcx, qcy, qcz, stage, idxms, svb, fbuf,
                  pends, pendd, sem):
        cid = lax.axis_index("c")
        sid = lax.axis_index("s")
        ii = lax.iota(jnp.int32, 16)

        def zero_stage_and_slice():
            z = ii.astype(jnp.float32) * 0.0
            for r in range(128):
                for k in range(kblk):
                    stage[r, pl.ds(k * 16, 16)] = z
            for j in range(trows // 128):
                pltpu.sync_copy(stage.at[pl.ds(0, 128)],
                                shS.at[pl.ds(sid * trows + j * 128, 128)])

        def process(cntv, lo):
            dvs = []
            for v in range(vb):
                maskv = ii + v * 16 < cntv
                svv = jnp.where(maskv, pends[pl.ds(v * 16, 16)], 0)
                dvs.append(jnp.where(maskv, pendd[pl.ds(v * 16, 16)], lo))
                svb[pl.ds(v * 16, 16)] = svv
            pltpu.async_copy(feats.at[svb], fbuf, sem).wait()
            for v in range(vb):
                maskv = ii + v * 16 < cntv
                dv = dvs[v]
                sx = plsc.load_gather(fbuf, [ii + v * 16, ii * 0 + in_ch])
                sy = plsc.load_gather(fbuf, [ii + v * 16, ii * 0 + (in_ch + 1)])
                sz = plsc.load_gather(fbuf, [ii + v * 16, ii * 0 + (in_ch + 2)])
                dloc = dv - lo
                dx = plsc.load_gather(qcx, [dloc])
                dy = plsc.load_gather(qcy, [dloc])
                dz = plsc.load_gather(qcz, [dloc])
                pxv = (sx - dx) * _INV_RADIUS
                pyv = (sy - dy) * _INV_RADIUS
                pzv = (sz - dz) * _INV_RADIUS
                r2 = pxv * pxv + pyv * pyv + pzv * pzv
                win = 1.0 - r2
                win = win * win * win
                win = jnp.clip(win, 0.0, 1.0)
                win = jnp.where(maskv, win, 0.0)
                r2c = jnp.maximum(r2, 1e-16)
                norm2 = r2c * _rsqrt(r2c)
                ninf = jnp.maximum(jnp.maximum(jnp.abs(pxv), jnp.abs(pyv)),
                                   jnp.abs(pzv))
                scale = jnp.where(ninf > 1e-8,
                                  norm2 / jnp.maximum(ninf, 1e-8), 0.0)
                g0 = []
                fr = []
                for pv in (pxv, pyv, pzv):
                    g = ((jnp.clip(pv * scale, -1.0, 1.0) + 1.0)
                         * (0.5 * (_KS - 1)))
                    gi = jnp.clip(g.astype(jnp.int32), 0, _KS - 2)
                    g0.append(gi)
                    fr.append(g - gi.astype(jnp.float32))
                rowbase = dloc * _NBIN + g0[0] * 16 + g0[1] * 4 + g0[2]
                wx = (win * (1.0 - fr[0]), win * fr[0])
                wy = (1.0 - fr[1], fr[1])
                wz = (1.0 - fr[2], fr[2])
                wcs = []
                for c, (cx, cy, cz) in enumerate(_CORNERS):
                    wcs.append(wx[cx] * wy[cy] * wz[cz])
                    idxms[v][pl.ds(c * 16, 16)] = (rowbase
                                                   + (cx * 16 + cy * 4 + cz))
                for e in range(16):
                    fe = [fbuf[v * 16 + e, pl.ds(k * 16, 16)]
                          for k in range(kblk)]
                    for c in range(8):
                        w_sc = wcs[c][e]
                        for k in range(kblk):
                            stage[v * 128 + c * 16 + e,
                                  pl.ds(k * 16, 16)] = fe[k] * w_sc
            for v in range(vb):
                pltpu.sync_copy(stage.at[pl.ds(v * 128, 128)],
                                shS.at[idxms[v]], add=True)
            for a in (pends, pendd):
                a[pl.ds(0, 16)] = a[pl.ds(ebatch, 16)]
                a[pl.ds(16, 16)] = a[pl.ds(ebatch + 16, 16)]
            return cntv - ebatch

        zero_stage_and_slice()
        plsc.subcore_barrier()

        def chunk_body(ch, carry):
            lo = cid * _HALF + ch * _CHUNK_DST
            pltpu.sync_copy(qx.at[pl.ds(lo, _CHUNK_DST)], qcx)
            pltpu.sync_copy(qy.at[pl.ds(lo, _CHUNK_DST)], qcy)
            pltpu.sync_copy(qz.at[pl.ds(lo, _CHUNK_DST)], qcz)

            def block_body(b, cnt):
                base = sid * ept + b * _EBLK
                pltpu.sync_copy(esrc.at[pl.ds(base, _EBLK)], ebs)
                pltpu.sync_copy(edst.at[pl.ds(base, _EBLK)], ebd)

                def scan_body(i, cnt):
                    sv = ebs[pl.ds(i * 16, 16)]
                    dv = ebd[pl.ds(i * 16, 16)]
                    m = (dv >= lo) & (dv < lo + _CHUNK_DST)
                    plsc.store_compressed(pends.at[pl.ds(cnt, 16)], sv, mask=m)
                    plsc.store_compressed(pendd.at[pl.ds(cnt, 16)], dv, mask=m)
                    cnt = cnt + plsc.all_reduce_population_count(m)[0]
                    return lax.cond(cnt >= ebatch,
                                    lambda c: process(ebatch, lo)
                                    + (c - ebatch),
                                    lambda c: c, cnt)

                return lax.fori_loop(0, nvec, scan_body, cnt)

            cnt = lax.fori_loop(0, nblk, block_body, 0)
            cnt = lax.cond(cnt > 0, lambda c: process(c, lo) * 0,
                           lambda c: c, cnt)
            plsc.subcore_barrier()
            pltpu.sync_copy(
                shS.at[pl.ds(sid * trows, trows)],
                s_hbm.at[pl.ds(lo * _NBIN + sid * trows, trows)])
            zero_stage_and_slice()
            plsc.subcore_barrier()
            return carry

        lax.fori_loop(0, _NCHUNK, chunk_body, 0)

    return sc_kernel


_sc_fluid16 = _make_sc_scatter(320000, 16, 32)
_sc_obst16 = _make_sc_scatter(160000, 16, 32)
_sc_fluid96 = _make_sc_scatter(320000, 96, 16)
_sc_fluid64 = _make_sc_scatter(320000, 64, 32)


def _tc_layer0(S0o, W0o, S0f, W0f, ff, D0w, bcat):
    bn = 256
    ko = W0o.shape[0]
    kf = W0f.shape[0]

    def body(so_ref, wo_ref, sf_ref, wf_ref, ff_ref, dw_ref, b_ref, o_ref):
        a0o = jnp.dot(so_ref[...], wo_ref[...],
                      preferred_element_type=jnp.float32)
        a0f = jnp.dot(sf_ref[...], wf_ref[...],
                      preferred_element_type=jnp.float32)
        a0d = jnp.dot(ff_ref[...], dw_ref[...],
                      preferred_element_type=jnp.float32)
        x = jnp.concatenate([a0o, a0f, a0d], axis=1) + b_ref[...]
        o_ref[...] = jnp.maximum(x, 0.0)

    return pl.pallas_call(
        body,
        grid=(_NPAD // bn,),
        in_specs=[pl.BlockSpec((bn, ko), lambda i: (i, 0)),
                  pl.BlockSpec((ko, 32), lambda i: (0, 0)),
                  pl.BlockSpec((bn, kf), lambda i: (i, 0)),
                  pl.BlockSpec((kf, 32), lambda i: (0, 0)),
                  pl.BlockSpec((bn, 16), lambda i: (i, 0)),
                  pl.BlockSpec((16, 32), lambda i: (0, 0)),
                  pl.BlockSpec((1, 96), lambda i: (0, 0))],
        out_specs=pl.BlockSpec((bn, 96), lambda i: (i, 0)),
        out_shape=jax.ShapeDtypeStruct((_NPAD, 96), jnp.float32),
    )(S0o, W0o, S0f, W0f, ff, D0w, bcat)


def _tc_layer(S2d, Wflat, h, Dw, bias, res, want_relu):
    bn = 256
    k = Wflat.shape[0]
    kh = Dw.shape[0]
    oc = Wflat.shape[1]
    n_out = 2 if want_relu else 1

    def body(*refs):
        if res is not None:
            s_ref, w_ref, h_ref, dw_ref, b_ref, r_ref = refs[:6]
            outs = refs[6:]
        else:
            s_ref, w_ref, h_ref, dw_ref, b_ref = refs[:5]
            outs = refs[5:]
        x = jnp.dot(s_ref[...], w_ref[...],
                    preferred_element_type=jnp.float32)
        x = x + jnp.dot(h_ref[...], dw_ref[...],
                        preferred_element_type=jnp.float32)
        x = x + b_ref[...]
        if res is not None:
            x = x + r_ref[...]
        outs[0][...] = x
        if want_relu:
            outs[1][...] = jnp.maximum(x, 0.0)

    in_specs = [pl.BlockSpec((bn, k), lambda i: (i, 0)),
                pl.BlockSpec((k, oc), lambda i: (0, 0)),
                pl.BlockSpec((bn, kh), lambda i: (i, 0)),
                pl.BlockSpec((kh, oc), lambda i: (0, 0)),
                pl.BlockSpec((1, oc), lambda i: (0, 0))]
    args = [S2d, Wflat, h, Dw, bias]
    if res is not None:
        in_specs.append(pl.BlockSpec((bn, oc), lambda i: (i, 0)))
        args.append(res)
    out_specs = [pl.BlockSpec((bn, oc), lambda i: (i, 0))] * n_out
    out_shape = [jax.ShapeDtypeStruct((_NPAD, oc), jnp.float32)] * n_out
    if n_out == 1:
        out_specs, out_shape = out_specs[0], out_shape[0]
    return pl.pallas_call(
        body,
        grid=(_NPAD // bn,),
        in_specs=in_specs,
        out_specs=out_specs,
        out_shape=out_shape,
    )(*args)


def kernel(pos, vel, box, box_feats, edge_index_fluid, edge_index_obstacle,
           W0f, b0f, W0o, b0o, D0w, D0b, W1, b1, D1w, D1b,
           W2, b2, D2w, D2b, W3, b3, D3w, D3b):
    n = pos.shape[0]
    m = box.shape[0]
    f32 = jnp.float32

    posp = jnp.pad(pos, ((0, _NPAD - n), (0, 0)))
    qx, qy, qz = posp[:, 0], posp[:, 1], posp[:, 2]
    post16 = jnp.pad(posp, ((0, 0), (0, 13)))
    boxt16 = jnp.pad(box, ((0, _NPAD - m), (0, 13)))
    sf, df = edge_index_fluid[0], edge_index_fluid[1]
    so, do = edge_index_obstacle[0], edge_index_obstacle[1]

    ff = jnp.concatenate([jnp.ones_like(vel[:, 0:1]), vel], axis=-1)
    ff16 = jnp.pad(ff, ((0, _NPAD - n), (0, 12)))
    bf16 = jnp.pad(box_feats, ((0, _NPAD - m), (0, 13)))
    ff_aug = jnp.concatenate([ff16, post16], axis=1)
    bf_aug = jnp.concatenate([bf16, boxt16], axis=1)

    W0f_p = jnp.pad(W0f.reshape(_NBIN, 4, 32),
                    ((0, 0), (0, 12), (0, 0))).reshape(_NBIN * 16, 32)
    W0o_p = jnp.pad(W0o.reshape(_NBIN, 3, 32),
                    ((0, 0), (0, 13), (0, 0))).reshape(_NBIN * 16, 32)
    W1f = W1.reshape(_NBIN * 96, 64)
    W2f = W2.reshape(_NBIN * 64, 64)

    S0f = _sc_fluid16(sf, df, qx, qy, qz, ff_aug)
    S0o = _sc_obst16(so, do, qx, qy, qz, bf_aug)
    bcat = jnp.concatenate([b0o, b0f, D0b]).reshape(1, 96)
    h1 = _tc_layer0(S0o.reshape(_NPAD, _NBIN * 16), W0o_p,
                    S0f.reshape(_NPAD, _NBIN * 16), W0f_p,
                    ff16, jnp.pad(D0w, ((0, 12), (0, 0))), bcat)

    h1_aug = jnp.concatenate([h1, post16], axis=1)
    S1 = _sc_fluid96(sf, df, qx, qy, qz, h1_aug)
    out1, h2 = _tc_layer(S1.reshape(_NPAD, _NBIN * 96), W1f, h1, D1w,
                         (b1 + D1b).reshape(1, 64), None, True)

    h2_aug = jnp.concatenate([h2, post16], axis=1)
    S2 = _sc_fluid64(sf, df, qx, qy, qz, h2_aug)
    out2 = _tc_layer(S2.reshape(_NPAD, _NBIN * 64), W2f, h2, D2w,
                     (b2 + D2b).reshape(1, 64), out1, False)
    return out2[:n]


# speedup vs baseline: 4.9030x; 1.2276x over previous
import functools

import jax
import jax.numpy as jnp
from jax import lax
from jax.experimental import pallas as pl
from jax.experimental.pallas import tpu as pltpu
from jax.experimental.pallas import tpu_sc as plsc

_KS = 4
_NBIN = 64
_EXTENT = float(1.5 * 6 * 0.025)
_INV_RADIUS = 2.0 / _EXTENT
_NPAD = 10240
_NTILE = 16
_HALF = 5120
_EBLK = 2000

_CP = pltpu.CompilerParams(needs_layout_passes=False,
                           use_tc_tiling_on_sc=False)

_CORNERS = [(cx, cy, cz) for cx in (0, 1) for cy in (0, 1) for cz in (0, 1)]


def _rsqrt(x):
    b = plsc.bitcast(x, jnp.int32)
    y = plsc.bitcast(jnp.int32(0x5F3759DF) - (b >> 1), jnp.float32)
    y = y * (1.5 - 0.5 * x * y * y)
    y = y * (1.5 - 0.5 * x * y * y)
    y = y * (1.5 - 0.5 * x * y * y)
    return y


def _make_sc_scatter(n_edges, in_ch, ebatch, cdst):
    assert in_ch % 16 == 0 and ebatch % 16 == 0
    kblk = in_ch // 16
    acols = in_ch + 16
    vb = ebatch // 16
    pcap = ebatch + 32
    ept = n_edges // _NTILE
    nblk = ept // _EBLK
    nvec = _EBLK // 16
    assert ept * _NTILE == n_edges and nblk * _EBLK == ept and nvec * 16 == _EBLK
    srows = cdst * _NBIN
    trows = srows // _NTILE
    nchunk = _HALF // cdst
    assert nchunk * cdst == _HALF
    mesh = plsc.VectorSubcoreMesh(core_axis_name="c", subcore_axis_name="s")

    @functools.partial(
        pl.kernel, mesh=mesh, compiler_params=_CP,
        out_type=jax.ShapeDtypeStruct((_NPAD * _NBIN, in_ch), jnp.float32),
        scratch_types=[
            pltpu.VMEM_SHARED((srows, in_ch), jnp.float32),
            pltpu.VMEM((_EBLK,), jnp.int32),
            pltpu.VMEM((_EBLK,), jnp.int32),
            pltpu.VMEM((cdst,), jnp.float32),
            pltpu.VMEM((cdst,), jnp.float32),
            pltpu.VMEM((cdst,), jnp.float32),
            pltpu.VMEM((vb * 128, in_ch), jnp.float32),
            [pltpu.VMEM((128,), jnp.int32) for _ in range(vb)],
            pltpu.VMEM((ebatch,), jnp.int32),
            pltpu.VMEM((ebatch, acols), jnp.float32),
            pltpu.VMEM((pcap,), jnp.int32),
            pltpu.VMEM((pcap,), jnp.int32),
            pltpu.SemaphoreType.DMA,
        ],
    )
    def sc_kernel(esrc, edst, qx, qy, qz, feats, s_hbm,
                  shS, ebs, ebd, qcx, qcy, qcz, stage, idxms, svb, fbuf,
                  pends, pendd, sem):
        cid = lax.axis_index("c")
        sid = lax.axis_index("s")
        ii = lax.iota(jnp.int32, 16)

        def zero_stage_and_slice():
            z = ii.astype(jnp.float32) * 0.0
            for r in range(128):
                for k in range(kblk):
                    stage[r, pl.ds(k * 16, 16)] = z
            for j in range(trows // 128):
                pltpu.sync_copy(stage.at[pl.ds(0, 128)],
                                shS.at[pl.ds(sid * trows + j * 128, 128)])

        def process(cntv, lo):
            dvs = []
            for v in range(vb):
                maskv = ii + v * 16 < cntv
                svv = jnp.where(maskv, pends[pl.ds(v * 16, 16)], 0)
                dvs.append(jnp.where(maskv, pendd[pl.ds(v * 16, 16)], lo))
                svb[pl.ds(v * 16, 16)] = svv
            pltpu.async_copy(feats.at[svb], fbuf, sem).wait()
            for v in range(vb):
                maskv = ii + v * 16 < cntv
                dv = dvs[v]
                sx = plsc.load_gather(fbuf, [ii + v * 16, ii * 0 + in_ch])
                sy = plsc.load_gather(fbuf, [ii + v * 16, ii * 0 + (in_ch + 1)])
                sz = plsc.load_gather(fbuf, [ii + v * 16, ii * 0 + (in_ch + 2)])
                dloc = dv - lo
                dx = plsc.load_gather(qcx, [dloc])
                dy = plsc.load_gather(qcy, [dloc])
                dz = plsc.load_gather(qcz, [dloc])
                pxv = (sx - dx) * _INV_RADIUS
                pyv = (sy - dy) * _INV_RADIUS
                pzv = (sz - dz) * _INV_RADIUS
                r2 = pxv * pxv + pyv * pyv + pzv * pzv
                win = 1.0 - r2
                win = win * win * win
                win = jnp.clip(win, 0.0, 1.0)
                win = jnp.where(maskv, win, 0.0)
                r2c = jnp.maximum(r2, 1e-16)
                norm2 = r2c * _rsqrt(r2c)
                ninf = jnp.maximum(jnp.maximum(jnp.abs(pxv), jnp.abs(pyv)),
                                   jnp.abs(pzv))
                scale = jnp.where(ninf > 1e-8,
                                  norm2 / jnp.maximum(ninf, 1e-8), 0.0)
                g0 = []
                fr = []
                for pv in (pxv, pyv, pzv):
                    g = ((jnp.clip(pv * scale, -1.0, 1.0) + 1.0)
                         * (0.5 * (_KS - 1)))
                    gi = jnp.clip(g.astype(jnp.int32), 0, _KS - 2)
                    g0.append(gi)
                    fr.append(g - gi.astype(jnp.float32))
                rowbase = dloc * _NBIN + g0[0] * 16 + g0[1] * 4 + g0[2]
                wx = (win * (1.0 - fr[0]), win * fr[0])
                wy = (1.0 - fr[1], fr[1])
                wz = (1.0 - fr[2], fr[2])
                wcs = []
                for c, (cx, cy, cz) in enumerate(_CORNERS):
                    wcs.append(wx[cx] * wy[cy] * wz[cz])
                    idxms[v][pl.ds(c * 16, 16)] = (rowbase
                                                   + (cx * 16 + cy * 4 + cz))
                for e in range(16):
                    fe = [fbuf[v * 16 + e, pl.ds(k * 16, 16)]
                          for k in range(kblk)]
                    for c in range(8):
                        w_sc = wcs[c][e]
                        for k in range(kblk):
                            stage[v * 128 + c * 16 + e,
                                  pl.ds(k * 16, 16)] = fe[k] * w_sc
            for v in range(vb):
                pltpu.sync_copy(stage.at[pl.ds(v * 128, 128)],
                                shS.at[idxms[v]], add=True)
            for a in (pends, pendd):
                a[pl.ds(0, 16)] = a[pl.ds(ebatch, 16)]
                a[pl.ds(16, 16)] = a[pl.ds(ebatch + 16, 16)]
            return cntv - ebatch

        zero_stage_and_slice()
        plsc.subcore_barrier()

        def chunk_body(ch, carry):
            lo = cid * _HALF + ch * cdst
            pltpu.sync_copy(qx.at[pl.ds(lo, cdst)], qcx)
            pltpu.sync_copy(qy.at[pl.ds(lo, cdst)], qcy)
            pltpu.sync_copy(qz.at[pl.ds(lo, cdst)], qcz)

            def block_body(b, cnt):
                base = sid * ept + b * _EBLK
                pltpu.sync_copy(esrc.at[pl.ds(base, _EBLK)], ebs)
                pltpu.sync_copy(edst.at[pl.ds(base, _EBLK)], ebd)

                def scan_body(i, cnt):
                    sv = ebs[pl.ds(i * 16, 16)]
                    dv = ebd[pl.ds(i * 16, 16)]
                    m = (dv >= lo) & (dv < lo + cdst)
                    plsc.store_compressed(pends.at[pl.ds(cnt, 16)], sv, mask=m)
                    plsc.store_compressed(pendd.at[pl.ds(cnt, 16)], dv, mask=m)
                    cnt = cnt + plsc.all_reduce_population_count(m)[0]
                    return lax.cond(cnt >= ebatch,
                                    lambda c: process(ebatch, lo)
                                    + (c - ebatch),
                                    lambda c: c, cnt)

                return lax.fori_loop(0, nvec, scan_body, cnt)

            cnt = lax.fori_loop(0, nblk, block_body, 0)
            cnt = lax.cond(cnt > 0, lambda c: process(c, lo) * 0,
                           lambda c: c, cnt)
            plsc.subcore_barrier()
            pltpu.sync_copy(
                shS.at[pl.ds(sid * trows, trows)],
                s_hbm.at[pl.ds(lo * _NBIN + sid * trows, trows)])
            zero_stage_and_slice()
            plsc.subcore_barrier()
            return carry

        lax.fori_loop(0, nchunk, chunk_body, 0)

    return sc_kernel


_sc_fluid16 = _make_sc_scatter(320000, 16, 32, 1024)
_sc_obst16 = _make_sc_scatter(160000, 16, 32, 1024)
_sc_fluid96 = _make_sc_scatter(320000, 96, 16, 256)
_sc_fluid64 = _make_sc_scatter(320000, 64, 32, 320)


def _tc_layer0(S0o, W0o, S0f, W0f, ff, D0w, bcat):
    bn = 256
    ko = W0o.shape[0]
    kf = W0f.shape[0]

    def body(so_ref, wo_ref, sf_ref, wf_ref, ff_ref, dw_ref, b_ref, o_ref):
        a0o = jnp.dot(so_ref[...], wo_ref[...],
                      preferred_element_type=jnp.float32)
        a0f = jnp.dot(sf_ref[...], wf_ref[...],
                      preferred_element_type=jnp.float32)
        a0d = jnp.dot(ff_ref[...], dw_ref[...],
                      preferred_element_type=jnp.float32)
        x = jnp.concatenate([a0o, a0f, a0d], axis=1) + b_ref[...]
        o_ref[...] = jnp.maximum(x, 0.0)

    return pl.pallas_call(
        body,
        grid=(_NPAD // bn,),
        in_specs=[pl.BlockSpec((bn, ko), lambda i: (i, 0)),
                  pl.BlockSpec((ko, 32), lambda i: (0, 0)),
                  pl.BlockSpec((bn, kf), lambda i: (i, 0)),
                  pl.BlockSpec((kf, 32), lambda i: (0, 0)),
                  pl.BlockSpec((bn, 16), lambda i: (i, 0)),
                  pl.BlockSpec((16, 32), lambda i: (0, 0)),
                  pl.BlockSpec((1, 96), lambda i: (0, 0))],
        out_specs=pl.BlockSpec((bn, 96), lambda i: (i, 0)),
        out_shape=jax.ShapeDtypeStruct((_NPAD, 96), jnp.float32),
    )(S0o, W0o, S0f, W0f, ff, D0w, bcat)


def _tc_layer(S2d, Wflat, h, Dw, bias, res, want_relu):
    bn = 256
    k = Wflat.shape[0]
    kh = Dw.shape[0]
    oc = Wflat.shape[1]
    n_out = 2 if want_relu else 1

    def body(*refs):
        if res is not None:
            s_ref, w_ref, h_ref, dw_ref, b_ref, r_ref = refs[:6]
            outs = refs[6:]
        else:
            s_ref, w_ref, h_ref, dw_ref, b_ref = refs[:5]
            outs = refs[5:]
        x = jnp.dot(s_ref[...], w_ref[...],
                    preferred_element_type=jnp.float32)
        x = x + jnp.dot(h_ref[...], dw_ref[...],
                        preferred_element_type=jnp.float32)
        x = x + b_ref[...]
        if res is not None:
            x = x + r_ref[...]
        outs[0][...] = x
        if want_relu:
            outs[1][...] = jnp.maximum(x, 0.0)

    in_specs = [pl.BlockSpec((bn, k), lambda i: (i, 0)),
                pl.BlockSpec((k, oc), lambda i: (0, 0)),
                pl.BlockSpec((bn, kh), lambda i: (i, 0)),
                pl.BlockSpec((kh, oc), lambda i: (0, 0)),
                pl.BlockSpec((1, oc), lambda i: (0, 0))]
    args = [S2d, Wflat, h, Dw, bias]
    if res is not None:
        in_specs.append(pl.BlockSpec((bn, oc), lambda i: (i, 0)))
        args.append(res)
    out_specs = [pl.BlockSpec((bn, oc), lambda i: (i, 0))] * n_out
    out_shape = [jax.ShapeDtypeStruct((_NPAD, oc), jnp.float32)] * n_out
    if n_out == 1:
        out_specs, out_shape = out_specs[0], out_shape[0]
    return pl.pallas_call(
        body,
        grid=(_NPAD // bn,),
        in_specs=in_specs,
        out_specs=out_specs,
        out_shape=out_shape,
    )(*args)


def kernel(pos, vel, box, box_feats, edge_index_fluid, edge_index_obstacle,
           W0f, b0f, W0o, b0o, D0w, D0b, W1, b1, D1w, D1b,
           W2, b2, D2w, D2b, W3, b3, D3w, D3b):
    n = pos.shape[0]
    m = box.shape[0]
    f32 = jnp.float32

    posp = jnp.pad(pos, ((0, _NPAD - n), (0, 0)))
    qx, qy, qz = posp[:, 0], posp[:, 1], posp[:, 2]
    post16 = jnp.pad(posp, ((0, 0), (0, 13)))
    boxt16 = jnp.pad(box, ((0, _NPAD - m), (0, 13)))
    sf, df = edge_index_fluid[0], edge_index_fluid[1]
    so, do = edge_index_obstacle[0], edge_index_obstacle[1]

    ff = jnp.concatenate([jnp.ones_like(vel[:, 0:1]), vel], axis=-1)
    ff16 = jnp.pad(ff, ((0, _NPAD - n), (0, 12)))
    bf16 = jnp.pad(box_feats, ((0, _NPAD - m), (0, 13)))
    ff_aug = jnp.concatenate([ff16, post16], axis=1)
    bf_aug = jnp.concatenate([bf16, boxt16], axis=1)

    W0f_p = jnp.pad(W0f.reshape(_NBIN, 4, 32),
                    ((0, 0), (0, 12), (0, 0))).reshape(_NBIN * 16, 32)
    W0o_p = jnp.pad(W0o.reshape(_NBIN, 3, 32),
                    ((0, 0), (0, 13), (0, 0))).reshape(_NBIN * 16, 32)
    W1f = W1.reshape(_NBIN * 96, 64)
    W2f = W2.reshape(_NBIN * 64, 64)

    S0f = _sc_fluid16(sf, df, qx, qy, qz, ff_aug)
    S0o = _sc_obst16(so, do, qx, qy, qz, bf_aug)
    bcat = jnp.concatenate([b0o, b0f, D0b]).reshape(1, 96)
    h1 = _tc_layer0(S0o.reshape(_NPAD, _NBIN * 16), W0o_p,
                    S0f.reshape(_NPAD, _NBIN * 16), W0f_p,
                    ff16, jnp.pad(D0w, ((0, 12), (0, 0))), bcat)

    h1_aug = jnp.concatenate([h1, post16], axis=1)
    S1 = _sc_fluid96(sf, df, qx, qy, qz, h1_aug)
    out1, h2 = _tc_layer(S1.reshape(_NPAD, _NBIN * 96), W1f, h1, D1w,
                         (b1 + D1b).reshape(1, 64), None, True)

    h2_aug = jnp.concatenate([h2, post16], axis=1)
    S2 = _sc_fluid64(sf, df, qx, qy, qz, h2_aug)
    out2 = _tc_layer(S2.reshape(_NPAD, _NBIN * 64), W2f, h2, D2w,
                     (b2 + D2b).reshape(1, 64), out1, False)
    return out2[:n]
